# baseline (speedup 1.0000x reference)
import jax
import jax.numpy as jnp
from jax import lax
from jax.experimental import pallas as pl
from jax.experimental.pallas import tpu as pltpu
from jax.experimental.pallas import tpu_sc as plsc

_N_POINTS = 16384
_N_IN = 100000
_D_FEAT = 128

_NC = 2
_NS = 16
_NW = _NC * _NS
_ROWS_PER_W = _N_POINTS // _NW
_FC = 128
_NFC = _ROWS_PER_W // _FC
_G = 16


def _body(coords_hbm, features_hbm, colors_hbm, normals_hbm, idx_hbm,
          out_c, out_f, out_col, out_n,
          idx_v, fa, fb, raw,
          sem_f0, sem_f1, sem_w0, sem_w1, sem_s, sem_sw):
    wid = lax.axis_index("s") * _NC + lax.axis_index("c")
    base = wid * _ROWS_PER_W

    pltpu.sync_copy(idx_hbm.at[pl.ds(base, _ROWS_PER_W)], idx_v)

    fbufs = (fa, fb)
    fsems = (sem_f0, sem_f1)
    wsems = (sem_w0, sem_w1)

    def fgather(c):
        return pltpu.async_copy(
            features_hbm.at[idx_v.at[pl.ds(c * _FC, _FC)]],
            fbufs[c % 2], fsems[c % 2])

    def fwrite(c):
        return pltpu.async_copy(
            fbufs[c % 2], out_f.at[pl.ds(base + c * _FC, _FC)], wsems[c % 2])

    def issue_rows(tbl):
        def group_body(g, carry):
            s = g * _G
            rows = idx_v[pl.ds(s, _G)]
            for lane in range(_G):
                pltpu.async_copy(tbl.at[pl.ds(rows[lane], 1)],
                                 raw.at[pl.ds(s + lane, 1)], sem_s)
            return carry
        lax.fori_loop(0, _ROWS_PER_W // _G, group_body, 0)

    def drain_rows(tbl):
        pltpu.make_async_copy(tbl.at[pl.ds(0, _ROWS_PER_W)], raw,
                              sem_s).wait()

    g0 = fgather(0)
    g1 = fgather(1)

    issue_rows(coords_hbm)
    g0.wait()
    w0 = fwrite(0)
    drain_rows(coords_hbm)
    sw0 = pltpu.async_copy(raw, out_c.at[pl.ds(base, _ROWS_PER_W)], sem_sw)
    g1.wait()
    w1 = fwrite(1)
    sw0.wait()

    issue_rows(colors_hbm)
    w0.wait()
    g2 = fgather(2)
    drain_rows(colors_hbm)
    sw1 = pltpu.async_copy(raw, out_col.at[pl.ds(base, _ROWS_PER_W)], sem_sw)
    w1.wait()
    g3 = fgather(3)
    sw1.wait()

    issue_rows(normals_hbm)
    g2.wait()
    w2 = fwrite(2)
    drain_rows(normals_hbm)
    sw2 = pltpu.async_copy(raw, out_n.at[pl.ds(base, _ROWS_PER_W)], sem_sw)
    g3.wait()
    w3 = fwrite(3)

    w2.wait()
    w3.wait()
    sw2.wait()


@jax.jit
def _downsample(coords, features, colors, normals, idx32):
    f32 = jnp.float32
    run = pl.kernel(
        _body,
        out_type=(
            jax.ShapeDtypeStruct((_N_POINTS, 3), f32),
            jax.ShapeDtypeStruct((_N_POINTS, _D_FEAT), f32),
            jax.ShapeDtypeStruct((_N_POINTS, 3), f32),
            jax.ShapeDtypeStruct((_N_POINTS, 3), f32),
        ),
        mesh=plsc.VectorSubcoreMesh(core_axis_name="c", subcore_axis_name="s"),
        compiler_params=pltpu.CompilerParams(needs_layout_passes=False),
        scratch_types=[
            pltpu.VMEM((_ROWS_PER_W,), jnp.int32),
            pltpu.VMEM((_FC, _D_FEAT), f32),
            pltpu.VMEM((_FC, _D_FEAT), f32),
            pltpu.VMEM((_ROWS_PER_W, 3), f32),
            pltpu.SemaphoreType.DMA,
            pltpu.SemaphoreType.DMA,
            pltpu.SemaphoreType.DMA,
            pltpu.SemaphoreType.DMA,
            pltpu.SemaphoreType.DMA,
            pltpu.SemaphoreType.DMA,
        ],
    )
    return run(coords, features, colors, normals, idx32)


def kernel(coords, features, colors, normals, idx):
    idx32 = idx.astype(jnp.int32)
    out_c, out_f, out_col, out_n = _downsample(coords, features, colors,
                                               normals, idx32)
    return (out_c, out_f, out_col, out_n)


# speedup vs baseline: 1.9013x; 1.9013x over previous
import jax
import jax.numpy as jnp
from jax import lax
from jax.experimental import pallas as pl
from jax.experimental.pallas import tpu as pltpu
from jax.experimental.pallas import tpu_sc as plsc

_N_POINTS = 16384
_N_IN = 100000
_N_IN_PAD = 100096
_D_FEAT = 128

_NC = 2
_NS = 16

_N_SMALL_W = 9
_N_FEAT_W = 16
_FEAT_W0 = _N_SMALL_W
_FROWS = _N_POINTS // _N_FEAT_W
_FC = 128
_NFC = _FROWS // _FC
_PHALF = _N_POINTS // 2
_RHALF = _N_IN_PAD // 2


def _body(*refs):
    small_in = refs[0:9]
    features_hbm, idx_hbm, idxf_hbm = refs[9:12]
    small_out = refs[12:21]
    out_f = refs[21]
    rowv, ibuf, obuf, fa, fb, fidx = refs[22:28]
    sem_f0, sem_f1, sem_w0, sem_w1 = refs[28:32]

    wid = lax.axis_index("s") * _NC + lax.axis_index("c")

    def small_job(src, dst):
        for h in range(2):
            pltpu.sync_copy(idxf_hbm.at[pl.ds(h * _PHALF, _PHALF)], ibuf)
            for p in range(2):
                lo = p * _RHALF
                pltpu.sync_copy(src.at[pl.ds(lo, _RHALF)], rowv)

                def group(g, carry, lo=lo):
                    s = g * 16
                    iv = plsc.bitcast(ibuf[pl.ds(s, 16)], jnp.int32)
                    loc = iv - lo
                    m = jnp.logical_and(loc >= 0, loc < _RHALF)
                    lc = jnp.clip(loc, 0, _RHALF - 1)
                    vals = plsc.load_gather(rowv, [lc], mask=m)
                    cur = obuf[pl.ds(s, 16)]
                    obuf[pl.ds(s, 16)] = jnp.where(m, vals, cur)
                    return carry

                lax.fori_loop(0, _PHALF // 16, group, 0)
            pltpu.sync_copy(obuf, dst.at[pl.ds(h * _PHALF, _PHALF)])

    for w in range(_N_SMALL_W):
        @pl.when(wid == w)
        def _(src=small_in[w], dst=small_out[w]):
            small_job(src, dst)

    @pl.when(jnp.logical_and(wid >= _FEAT_W0, wid < _FEAT_W0 + _N_FEAT_W))
    def _():
        fbase = (wid - _FEAT_W0) * _FROWS
        pltpu.sync_copy(idx_hbm.at[pl.ds(fbase, _FROWS)], fidx)
        fbufs = (fa, fb)
        fsems = (sem_f0, sem_f1)
        wsems = (sem_w0, sem_w1)

        def fgather(c):
            return pltpu.async_copy(
                features_hbm.at[fidx.at[pl.ds(c * _FC, _FC)]],
                fbufs[c % 2], fsems[c % 2])

        def fwrite(c):
            return pltpu.async_copy(
                fbufs[c % 2], out_f.at[pl.ds(fbase + c * _FC, _FC)],
                wsems[c % 2])

        gs = [fgather(0), fgather(1)]
        ws = []
        for c in range(_NFC):
            gs[c].wait()
            ws.append(fwrite(c))
            if c + 2 < _NFC:
                ws[c].wait()
                gs.append(fgather(c + 2))
        ws[_NFC - 2].wait()
        ws[_NFC - 1].wait()


@jax.jit
def _downsample(small_rows, features, idx32, idxf):
    f32 = jnp.float32
    vec_out = jax.ShapeDtypeStruct((_N_POINTS,), f32)
    run = pl.kernel(
        _body,
        out_type=(vec_out,) * 9 + (
            jax.ShapeDtypeStruct((_N_POINTS, _D_FEAT), f32),),
        mesh=plsc.VectorSubcoreMesh(core_axis_name="c", subcore_axis_name="s"),
        compiler_params=pltpu.CompilerParams(needs_layout_passes=False),
        scratch_types=[
            pltpu.VMEM((_RHALF,), f32),
            pltpu.VMEM((_PHALF,), f32),
            pltpu.VMEM((_PHALF,), f32),
            pltpu.VMEM((_FC, _D_FEAT), f32),
            pltpu.VMEM((_FC, _D_FEAT), f32),
            pltpu.VMEM((_FROWS,), jnp.int32),
            pltpu.SemaphoreType.DMA,
            pltpu.SemaphoreType.DMA,
            pltpu.SemaphoreType.DMA,
            pltpu.SemaphoreType.DMA,
        ],
    )
    return run(*small_rows, features, idx32, idxf)


def kernel(coords, features, colors, normals, idx):
    idx32 = idx.astype(jnp.int32)
    idxf = jax.lax.bitcast_convert_type(idx32, jnp.float32)
    pad = _N_IN_PAD - _N_IN
    small_rows = []
    for tbl in (coords, colors, normals):
        tT = jnp.pad(tbl.T, ((0, 0), (0, pad)))
        for comp in range(3):
            small_rows.append(tT[comp])
    outs = _downsample(tuple(small_rows), features, idx32, idxf)
    small_outs, out_f = outs[0:9], outs[9]
    out_c = jnp.stack(small_outs[0:3], axis=1)
    out_col = jnp.stack(small_outs[3:6], axis=1)
    out_n = jnp.stack(small_outs[6:9], axis=1)
    return (out_c, out_f, out_col, out_n)


# speedup vs baseline: 2.2996x; 1.2095x over previous
import jax
import jax.numpy as jnp
from jax import lax
from jax.experimental import pallas as pl
from jax.experimental.pallas import tpu as pltpu
from jax.experimental.pallas import tpu_sc as plsc

_N_POINTS = 16384
_N_IN = 100000
_N_IN_PAD = 100096
_D_FEAT = 128

_NC = 2
_NS = 16

_N_SMALL_W = 18
_FEAT_W0 = _N_SMALL_W
_N_FEAT_W = 32 - _N_SMALL_W
_FC = 128
_NCHUNKS = _N_POINTS // _FC
_FCHUNKS = (10, 10) + (9,) * 12
_FSTARTS = tuple(sum(_FCHUNKS[:i]) for i in range(_N_FEAT_W))
_PHALF = _N_POINTS // 2
_RHALF = _N_IN_PAD // 2
_FIDX_MAX = max(_FCHUNKS) * _FC


def _body(*refs):
    small_in = refs[0:9]
    features_hbm, idx_hbm = refs[9:11]
    small_out = refs[11:20]
    out_f = refs[20]
    rowv, ibuf, obuf, fa, fb, fidx = refs[21:27]
    sem_f0, sem_f1, sem_w0, sem_w1 = refs[27:31]

    wid = lax.axis_index("s") * _NC + lax.axis_index("c")

    def small_job(src, dst, h):
        pltpu.sync_copy(idx_hbm.at[pl.ds(h * _PHALF, _PHALF)], ibuf)
        for p in range(2):
            lo = p * _RHALF
            pltpu.sync_copy(src.at[pl.ds(lo, _RHALF)], rowv)

            def group(g, carry, lo=lo):
                s = g * 16
                iv = ibuf[pl.ds(s, 16)]
                loc = iv - lo
                m = jnp.logical_and(loc >= 0, loc < _RHALF)
                lc = jnp.clip(loc, 0, _RHALF - 1)
                vals = plsc.load_gather(rowv, [lc], mask=m)
                cur = obuf[pl.ds(s, 16)]
                obuf[pl.ds(s, 16)] = jnp.where(m, vals, cur)
                return carry

            lax.fori_loop(0, _PHALF // 16, group, 0)
        pltpu.sync_copy(obuf, dst.at[pl.ds(h * _PHALF, _PHALF)])

    for w in range(_N_SMALL_W):
        @pl.when(wid == w)
        def _(src=small_in[w // 2], dst=small_out[w // 2], h=w % 2):
            small_job(src, dst, h)

    def feat_job(start, nc):
        fbase = start * _FC
        frows = nc * _FC
        pltpu.sync_copy(idx_hbm.at[pl.ds(fbase, frows)], fidx.at[pl.ds(0, frows)])
        fbufs = (fa, fb)
        fsems = (sem_f0, sem_f1)
        wsems = (sem_w0, sem_w1)

        def fgather(c):
            return pltpu.async_copy(
                features_hbm.at[fidx.at[pl.ds(c * _FC, _FC)]],
                fbufs[c % 2], fsems[c % 2])

        def fwrite(c):
            return pltpu.async_copy(
                fbufs[c % 2], out_f.at[pl.ds(fbase + c * _FC, _FC)],
                wsems[c % 2])

        gs = [fgather(0), fgather(1)]
        ws = []
        for c in range(nc):
            gs[c].wait()
            ws.append(fwrite(c))
            if c + 2 < nc:
                ws[c].wait()
                gs.append(fgather(c + 2))
        ws[nc - 2].wait()
        ws[nc - 1].wait()

    for w in range(_N_FEAT_W):
        @pl.when(wid == _FEAT_W0 + w)
        def _(start=_FSTARTS[w], nc=_FCHUNKS[w]):
            feat_job(start, nc)


@jax.jit
def _downsample(small_rows, features, idx32):
    f32 = jnp.float32
    vec_out = jax.ShapeDtypeStruct((_N_POINTS,), f32)
    run = pl.kernel(
        _body,
        out_type=(vec_out,) * 9 + (
            jax.ShapeDtypeStruct((_N_POINTS, _D_FEAT), f32),),
        mesh=plsc.VectorSubcoreMesh(core_axis_name="c", subcore_axis_name="s"),
        compiler_params=pltpu.CompilerParams(needs_layout_passes=False),
        scratch_types=[
            pltpu.VMEM((_RHALF,), f32),
            pltpu.VMEM((_PHALF,), jnp.int32),
            pltpu.VMEM((_PHALF,), f32),
            pltpu.VMEM((_FC, _D_FEAT), f32),
            pltpu.VMEM((_FC, _D_FEAT), f32),
            pltpu.VMEM((_FIDX_MAX,), jnp.int32),
            pltpu.SemaphoreType.DMA,
            pltpu.SemaphoreType.DMA,
            pltpu.SemaphoreType.DMA,
            pltpu.SemaphoreType.DMA,
        ],
    )
    return run(*small_rows, features, idx32)


def kernel(coords, features, colors, normals, idx):
    idx32 = idx.astype(jnp.int32)
    pad = _N_IN_PAD - _N_IN
    small_rows = tuple(jnp.pad(tbl[:, c], (0, pad))
                       for tbl in (coords, colors, normals)
                       for c in range(3))
    outs = _downsample(small_rows, features, idx32)
    small_outs, out_f = outs[0:9], outs[9]
    out_c = jnp.stack(small_outs[0:3], axis=1)
    out_col = jnp.stack(small_outs[3:6], axis=1)
    out_n = jnp.stack(small_outs[6:9], axis=1)
    return (out_c, out_f, out_col, out_n)


# speedup vs baseline: 2.3355x; 1.0156x over previous
import jax
import jax.numpy as jnp
from jax import lax
from jax.experimental import pallas as pl
from jax.experimental.pallas import tpu as pltpu
from jax.experimental.pallas import tpu_sc as plsc

_N_POINTS = 16384
_N_IN = 100000
_D_FEAT = 128

_NC = 2
_NS = 16

_N_SMALL_W = 18
_FEAT_W0 = _N_SMALL_W
_N_FEAT_W = 32 - _N_SMALL_W
_FC = 128
_FCHUNKS = (10, 10) + (9,) * 12
_FSTARTS = tuple(sum(_FCHUNKS[:i]) for i in range(_N_FEAT_W))
_PHALF = _N_POINTS // 2


def _body(*refs):
    small_in = refs[0:9]
    features_hbm, idx_hbm = refs[9:11]
    small_out = refs[11:20]
    out_f = refs[20]
    sem_f0, sem_f1, sem_w0, sem_w1 = refs[21:25]

    wid = lax.axis_index("s") * _NC + lax.axis_index("c")

    def small_job(src, dst, h):
        def inner(rowv, ibuf, obuf):
            pltpu.sync_copy(idx_hbm.at[pl.ds(h * _PHALF, _PHALF)], ibuf)
            pltpu.sync_copy(src, rowv)

            def group(g, carry):
                s = g * 16
                iv = ibuf[pl.ds(s, 16)]
                obuf[pl.ds(s, 16)] = plsc.load_gather(rowv, [iv])
                return carry

            lax.fori_loop(0, _PHALF // 16, group, 0)
            pltpu.sync_copy(obuf, dst.at[pl.ds(h * _PHALF, _PHALF)])

        pl.run_scoped(inner,
                      pltpu.VMEM((_N_IN,), jnp.float32),
                      pltpu.VMEM((_PHALF,), jnp.int32),
                      pltpu.VMEM((_PHALF,), jnp.float32))

    for w in range(_N_SMALL_W):
        @pl.when(wid == w)
        def _(src=small_in[w // 2], dst=small_out[w // 2], h=w % 2):
            small_job(src, dst, h)

    def feat_job(start, nc):
        fbase = start * _FC
        frows = nc * _FC

        def inner(fa, fb, fidx):
            pltpu.sync_copy(idx_hbm.at[pl.ds(fbase, frows)], fidx)
            fbufs = (fa, fb)
            fsems = (sem_f0, sem_f1)
            wsems = (sem_w0, sem_w1)

            def fgather(c):
                return pltpu.async_copy(
                    features_hbm.at[fidx.at[pl.ds(c * _FC, _FC)]],
                    fbufs[c % 2], fsems[c % 2])

            def fwrite(c):
                return pltpu.async_copy(
                    fbufs[c % 2], out_f.at[pl.ds(fbase + c * _FC, _FC)],
                    wsems[c % 2])

            gs = [fgather(0), fgather(1)]
            ws = []
            for c in range(nc):
                gs[c].wait()
                ws.append(fwrite(c))
                if c + 2 < nc:
                    ws[c].wait()
                    gs.append(fgather(c + 2))
            ws[nc - 2].wait()
            ws[nc - 1].wait()

        pl.run_scoped(inner,
                      pltpu.VMEM((_FC, _D_FEAT), jnp.float32),
                      pltpu.VMEM((_FC, _D_FEAT), jnp.float32),
                      pltpu.VMEM((frows,), jnp.int32))

    for w in range(_N_FEAT_W):
        @pl.when(wid == _FEAT_W0 + w)
        def _(start=_FSTARTS[w], nc=_FCHUNKS[w]):
            feat_job(start, nc)


@jax.jit
def _downsample(small_rows, features, idx32):
    f32 = jnp.float32
    vec_out = jax.ShapeDtypeStruct((_N_POINTS,), f32)
    run = pl.kernel(
        _body,
        out_type=(vec_out,) * 9 + (
            jax.ShapeDtypeStruct((_N_POINTS, _D_FEAT), f32),),
        mesh=plsc.VectorSubcoreMesh(core_axis_name="c", subcore_axis_name="s"),
        compiler_params=pltpu.CompilerParams(needs_layout_passes=False),
        scratch_types=[
            pltpu.SemaphoreType.DMA,
            pltpu.SemaphoreType.DMA,
            pltpu.SemaphoreType.DMA,
            pltpu.SemaphoreType.DMA,
        ],
    )
    return run(*small_rows, features, idx32)


def kernel(coords, features, colors, normals, idx):
    idx32 = idx.astype(jnp.int32)
    small_rows = tuple(tbl[:, c]
                       for tbl in (coords, colors, normals)
                       for c in range(3))
    outs = _downsample(small_rows, features, idx32)
    small_outs, out_f = outs[0:9], outs[9]
    out_c = jnp.stack(small_outs[0:3], axis=1)
    out_col = jnp.stack(small_outs[3:6], axis=1)
    out_n = jnp.stack(small_outs[6:9], axis=1)
    return (out_c, out_f, out_col, out_n)
